# Initial kernel scaffold; baseline (speedup 1.0000x reference)
import jax
import jax.numpy as jnp
from jax.experimental import pallas as pl


def kernel(x, group_indices, weight, bias):
    raise NotImplementedError("write your pallas kernel here")


# speedup vs baseline: 5.4304x; 5.4304x over previous
import jax
import jax.numpy as jnp
from jax.experimental import pallas as pl
from jax.experimental.pallas import tpu as pltpu

_C = 128


def _offsets_kernel(gi_ref, out_ref):
    idx = gi_ref[...]
    lanes = jax.lax.broadcasted_iota(jnp.int32, (idx.shape[0], 128), 1)
    lt = (idx < lanes).astype(jnp.int32)
    out_ref[...] = jnp.sum(lt, axis=0, keepdims=True)


def _gemm_kernel(starts_ref, x_ref, w_ref, b_ref, out_ref):
    g = pl.program_id(0)

    @pl.when(g == 0)
    def _():
        out_ref[...] = jnp.zeros_like(out_ref)

    start = starts_ref[g]
    end = starts_ref[g + 1]
    t0 = start // _C
    t1 = (end + _C - 1) // _C
    w = w_ref[0]
    b = b_ref[0]

    def body(t, carry):
        base = pl.multiple_of(t * _C, _C)
        xb = x_ref[pl.ds(base, _C), :]
        y = jax.lax.dot_general(
            xb, w, (((1,), (1,)), ((), ())),
            preferred_element_type=jnp.float32,
        )
        y = y + b
        rows = base + jax.lax.broadcasted_iota(jnp.int32, (_C, 1), 0)
        mask = (rows >= start) & (rows < end)
        out_ref[pl.ds(base, _C), :] += jnp.where(mask, y, 0.0)
        return carry

    jax.lax.fori_loop(t0, t1, body, 0)


def kernel(x, group_indices, weight, bias):
    n, k = x.shape
    g, o, _ = weight.shape
    gi = group_indices.astype(jnp.int32).reshape(n, 1)
    counts = pl.pallas_call(
        _offsets_kernel,
        out_shape=jax.ShapeDtypeStruct((1, 128), jnp.int32),
    )(gi)
    starts = counts.reshape(128)[: g + 1]

    grid_spec = pltpu.PrefetchScalarGridSpec(
        num_scalar_prefetch=1,
        grid=(g,),
        in_specs=[
            pl.BlockSpec((n, k), lambda i, s: (0, 0)),
            pl.BlockSpec((1, o, k), lambda i, s: (i, 0, 0)),
            pl.BlockSpec((1, 1, o), lambda i, s: (i, 0, 0)),
        ],
        out_specs=pl.BlockSpec((n, o), lambda i, s: (0, 0)),
    )
    out = pl.pallas_call(
        _gemm_kernel,
        grid_spec=grid_spec,
        out_shape=jax.ShapeDtypeStruct((n, o), x.dtype),
    )(starts, x, weight, bias.reshape(g, 1, o))
    return out


# speedup vs baseline: 5.4514x; 1.0039x over previous
import jax
import jax.numpy as jnp
from jax.experimental import pallas as pl
from jax.experimental.pallas import tpu as pltpu

_C = 128


def _offsets_kernel(gi_ref, out_ref):
    idx = gi_ref[...]
    lanes = jax.lax.broadcasted_iota(jnp.int32, (idx.shape[0], 128), 1)
    lt = (idx < lanes).astype(jnp.int32)
    out_ref[...] = jnp.sum(lt, axis=0, keepdims=True)


def _gemm_kernel(starts_ref, x_ref, w_ref, b_ref, out_ref):
    g = pl.program_id(0)
    start = starts_ref[g]
    end = starts_ref[g + 1]
    t0 = start // _C
    t1 = (end + _C - 1) // _C
    w = w_ref[0]
    b = b_ref[0]

    def body(t, carry):
        base = pl.multiple_of(t * _C, _C)
        xb = x_ref[pl.ds(base, _C), :]
        y = jax.lax.dot_general(
            xb, w, (((1,), (1,)), ((), ())),
            preferred_element_type=jnp.float32,
        )
        y = y + b
        rows = base + jax.lax.broadcasted_iota(jnp.int32, (_C, 1), 0)
        mask = (rows >= start) & (rows < end)
        out_ref[pl.ds(base, _C), :] = jnp.where(
            mask, y, out_ref[pl.ds(base, _C), :]
        )
        return carry

    jax.lax.fori_loop(t0, t1, body, 0)


def kernel(x, group_indices, weight, bias):
    n, k = x.shape
    g, o, _ = weight.shape
    gi = group_indices.astype(jnp.int32).reshape(n, 1)
    counts = pl.pallas_call(
        _offsets_kernel,
        out_shape=jax.ShapeDtypeStruct((1, 128), jnp.int32),
    )(gi)
    starts = counts.reshape(128)[: g + 1]

    grid_spec = pltpu.PrefetchScalarGridSpec(
        num_scalar_prefetch=1,
        grid=(g,),
        in_specs=[
            pl.BlockSpec((n, k), lambda i, s: (0, 0)),
            pl.BlockSpec((1, o, k), lambda i, s: (i, 0, 0)),
            pl.BlockSpec((1, 1, o), lambda i, s: (i, 0, 0)),
        ],
        out_specs=pl.BlockSpec((n, o), lambda i, s: (0, 0)),
    )
    out = pl.pallas_call(
        _gemm_kernel,
        grid_spec=grid_spec,
        out_shape=jax.ShapeDtypeStruct((n, o), x.dtype),
    )(starts, x, weight, bias.reshape(g, 1, o))
    return out
